# baseline (speedup 1.0000x reference)
import functools
import math

import jax
import jax.numpy as jnp
import numpy as np
from jax import lax
from jax.experimental import pallas as pl
from jax.experimental.pallas import tpu as pltpu
from jax.experimental.pallas import tpu_sc as plsc

VOCAB = 1000000
SEQ_LEN = 2048
D_EMBED = 128
BATCH = 4
SCALE = math.sqrt(float(D_EMBED))

NUM_CORES = 2
NUM_SUBCORES = 16
NW = NUM_CORES * NUM_SUBCORES
B_TOTAL = BATCH * SEQ_LEN
B_PER_W = B_TOTAL // NW
W_PER_BATCH = SEQ_LEN // B_PER_W
N_CHUNKS = 8
C_ROWS = B_PER_W // N_CHUNKS
LANES = 16


_A_ROWS = 16
_B_ROWS = SEQ_LEN // _A_ROWS


def _pe_tc_body(out_ref):
    k = lax.broadcasted_iota(jnp.int32, (1, D_EMBED), 1)
    half = (k // 2).astype(jnp.float32)
    rate = jnp.exp(half * (-2.0 * math.log(10000.0) / float(D_EMBED)))
    phase = jnp.where(k % 2 == 0, 0.0, 0.5 * math.pi)

    a_ang = (lax.broadcasted_iota(jnp.int32, (_A_ROWS, D_EMBED), 0)
             .astype(jnp.float32) * float(_B_ROWS)) * rate + phase
    b_ang = (lax.broadcasted_iota(jnp.int32, (_B_ROWS, D_EMBED), 0)
             .astype(jnp.float32)) * rate
    sin_a, cos_a = jnp.sin(a_ang), jnp.cos(a_ang)
    sin_b, cos_b = jnp.sin(b_ang), jnp.cos(b_ang)
    pe = (sin_a[:, None, :] * cos_b[None, :, :]
          + cos_a[:, None, :] * sin_b[None, :, :])
    out_ref[...] = pe.reshape(SEQ_LEN * D_EMBED)


_pe_table_tc = pl.pallas_call(
    _pe_tc_body,
    out_shape=jax.ShapeDtypeStruct((SEQ_LEN * D_EMBED,), jnp.float32),
)


def _make_sc_kernel():
    mesh = plsc.VectorSubcoreMesh(
        core_axis_name="c", subcore_axis_name="s")

    @functools.partial(
        pl.kernel,
        mesh=mesh,
        out_type=jax.ShapeDtypeStruct((BATCH, SEQ_LEN, D_EMBED), jnp.float32),
        scratch_types=[
            pltpu.VMEM((B_PER_W,), jnp.int32),
            pltpu.VMEM((N_CHUNKS, C_ROWS, D_EMBED), jnp.float32),
            pltpu.VMEM((N_CHUNKS, C_ROWS, D_EMBED), jnp.float32),
            pltpu.VMEM((B_PER_W * D_EMBED,), jnp.float32),
            pltpu.VMEM_SHARED((4, B_PER_W * D_EMBED), jnp.float32),
        ] + [pltpu.SemaphoreType.DMA] * (2 * N_CHUNKS + 2),
    )
    def emb_kernel(table_hbm, idx_hbm, pe_hbm, out_hbm,
                   idx_v, rows_v, comp_v, pe_v, pe_spm, *sems):
        gsems = sems[:N_CHUNKS]
        psems = sems[N_CHUNKS:2 * N_CHUNKS]
        w_sem, l_sem = sems[2 * N_CHUNKS], sems[2 * N_CHUNKS + 1]
        s = lax.axis_index("s")
        core = lax.axis_index("c")
        wid = s * NUM_CORES + core
        b = wid // W_PER_BATCH
        col0 = lax.rem(wid, W_PER_BATCH) * B_PER_W
        pe_off = pl.multiple_of(col0 * D_EMBED, 8)

        @pl.when(s < 4)
        def _():
            pltpu.async_copy(
                pe_hbm.at[pl.ds(pe_off, B_PER_W * D_EMBED)], pe_spm.at[s],
                l_sem)

        pltpu.sync_copy(idx_hbm.at[b, pl.ds(col0, B_PER_W)], idx_v)
        gathers = [
            pltpu.async_copy(
                table_hbm.at[idx_v.at[pl.ds(c * C_ROWS, C_ROWS)]],
                rows_v.at[c],
                gsems[c],
            )
            for c in range(N_CHUNKS)
        ]

        @pl.when(s < 4)
        def _():
            pltpu.make_async_copy(
                pe_hbm.at[pl.ds(pe_off, B_PER_W * D_EMBED)], pe_spm.at[s],
                l_sem).wait()

        plsc.subcore_barrier()
        slot = lax.rem(s, 4)
        pe_cps = [
            pltpu.async_copy(
                pe_spm.at[slot, pl.ds(c * C_ROWS * D_EMBED,
                                      C_ROWS * D_EMBED)],
                pe_v.at[pl.ds(c * C_ROWS * D_EMBED, C_ROWS * D_EMBED)],
                psems[c],
            )
            for c in range(N_CHUNKS)
        ]
        writes = []
        for c in range(N_CHUNKS):
            gathers[c].wait()
            pe_cps[c].wait()

            def row_body(r, carry, c=c):
                pe_base = (c * C_ROWS + r) * D_EMBED
                for j in range(D_EMBED // LANES):
                    sl = pl.ds(j * LANES, LANES)
                    comp_v[c, r, sl] = (rows_v[c, r, sl] * SCALE
                                        + pe_v[pl.ds(pe_base + j * LANES,
                                                     LANES)])
                return carry

            lax.fori_loop(0, C_ROWS, row_body, 0)
            writes.append(pltpu.async_copy(
                comp_v.at[c],
                out_hbm.at[b, pl.ds(col0 + c * C_ROWS, C_ROWS)],
                w_sem,
            ))
        for w in writes:
            w.wait()

    return emb_kernel


_EMB_KERNEL = _make_sc_kernel()


def kernel(input, token_table):
    pe = _pe_table_tc()
    return _EMB_KERNEL(token_table, input, pe)


# speedup vs baseline: 1.0066x; 1.0066x over previous
import functools
import math

import jax
import jax.numpy as jnp
from jax import lax
from jax.experimental import pallas as pl
from jax.experimental.pallas import tpu as pltpu
from jax.experimental.pallas import tpu_sc as plsc

VOCAB = 1000000
SEQ_LEN = 2048
D_EMBED = 128
BATCH = 4
SCALE = math.sqrt(float(D_EMBED))

NUM_CORES = 2
NUM_SUBCORES = 16
NW = NUM_CORES * NUM_SUBCORES
B_TOTAL = BATCH * SEQ_LEN
B_PER_W = B_TOTAL // NW
W_PER_BATCH = SEQ_LEN // B_PER_W
N_CHUNKS = 4
C_ROWS = B_PER_W // N_CHUNKS
LANES = 16


_A_ROWS = 16
_B_ROWS = SEQ_LEN // _A_ROWS


def _pe_tc_body(out_ref):
    k = lax.broadcasted_iota(jnp.int32, (1, D_EMBED), 1)
    half = (k // 2).astype(jnp.float32)
    rate = jnp.exp(half * (-2.0 * math.log(10000.0) / float(D_EMBED)))
    phase = jnp.where(k % 2 == 0, 0.0, 0.5 * math.pi)

    a_ang = (lax.broadcasted_iota(jnp.int32, (_A_ROWS, D_EMBED), 0)
             .astype(jnp.float32) * float(_B_ROWS)) * rate + phase
    b_ang = (lax.broadcasted_iota(jnp.int32, (_B_ROWS, D_EMBED), 0)
             .astype(jnp.float32)) * rate
    sin_a, cos_a = jnp.sin(a_ang), jnp.cos(a_ang)
    sin_b, cos_b = jnp.sin(b_ang), jnp.cos(b_ang)
    pe = (sin_a[:, None, :] * cos_b[None, :, :]
          + cos_a[:, None, :] * sin_b[None, :, :])
    out_ref[...] = pe.reshape(SEQ_LEN * D_EMBED)


_pe_table_tc = pl.pallas_call(
    _pe_tc_body,
    out_shape=jax.ShapeDtypeStruct((SEQ_LEN * D_EMBED,), jnp.float32),
)


def _make_sc_kernel():
    mesh = plsc.VectorSubcoreMesh(
        core_axis_name="c", subcore_axis_name="s")

    @functools.partial(
        pl.kernel,
        mesh=mesh,
        out_type=jax.ShapeDtypeStruct((BATCH, SEQ_LEN, D_EMBED), jnp.float32),
        scratch_types=[
            pltpu.VMEM((B_PER_W,), jnp.int32),
            pltpu.VMEM((N_CHUNKS, C_ROWS, D_EMBED), jnp.float32),
            pltpu.VMEM((N_CHUNKS, C_ROWS, D_EMBED), jnp.float32),
            pltpu.VMEM((B_PER_W * D_EMBED,), jnp.float32),
            pltpu.VMEM_SHARED((4, B_PER_W * D_EMBED), jnp.float32),
        ] + [pltpu.SemaphoreType.DMA] * (2 * N_CHUNKS + 2),
    )
    def emb_kernel(table_hbm, idx_hbm, pe_hbm, out_hbm,
                   idx_v, rows_v, comp_v, pe_v, pe_spm, *sems):
        gsems = sems[:N_CHUNKS]
        psems = sems[N_CHUNKS:2 * N_CHUNKS]
        w_sem, l_sem = sems[2 * N_CHUNKS], sems[2 * N_CHUNKS + 1]
        s = lax.axis_index("s")
        core = lax.axis_index("c")
        wid = s * NUM_CORES + core
        b = wid // W_PER_BATCH
        col0 = lax.rem(wid, W_PER_BATCH) * B_PER_W
        pe_off = pl.multiple_of(col0 * D_EMBED, 8)

        @pl.when(s < 4)
        def _():
            pltpu.async_copy(
                pe_hbm.at[pl.ds(pe_off, B_PER_W * D_EMBED)], pe_spm.at[s],
                l_sem)

        pltpu.sync_copy(idx_hbm.at[b, pl.ds(col0, B_PER_W)], idx_v)
        gathers = [
            pltpu.async_copy(
                table_hbm.at[idx_v.at[pl.ds(c * C_ROWS, C_ROWS)]],
                rows_v.at[c],
                gsems[c],
            )
            for c in range(N_CHUNKS)
        ]

        @pl.when(s < 4)
        def _():
            pltpu.make_async_copy(
                pe_hbm.at[pl.ds(pe_off, B_PER_W * D_EMBED)], pe_spm.at[s],
                l_sem).wait()

        plsc.subcore_barrier()
        slot = lax.rem(s, 4)
        pe_cps = [
            pltpu.async_copy(
                pe_spm.at[slot, pl.ds(c * C_ROWS * D_EMBED,
                                      C_ROWS * D_EMBED)],
                pe_v.at[pl.ds(c * C_ROWS * D_EMBED, C_ROWS * D_EMBED)],
                psems[c],
            )
            for c in range(N_CHUNKS)
        ]
        writes = []
        for c in range(N_CHUNKS):
            gathers[c].wait()
            pe_cps[c].wait()

            def row_body(r, carry, c=c):
                pe_base = (c * C_ROWS + r) * D_EMBED
                for j in range(D_EMBED // LANES):
                    sl = pl.ds(j * LANES, LANES)
                    comp_v[c, r, sl] = (rows_v[c, r, sl] * SCALE
                                        + pe_v[pl.ds(pe_base + j * LANES,
                                                     LANES)])
                return carry

            lax.fori_loop(0, C_ROWS, row_body, 0)
            writes.append(pltpu.async_copy(
                comp_v.at[c],
                out_hbm.at[b, pl.ds(col0 + c * C_ROWS, C_ROWS)],
                w_sem,
            ))
        for w in writes:
            w.wait()

    return emb_kernel


_EMB_KERNEL = _make_sc_kernel()


def kernel(input, token_table):
    pe = _pe_table_tc()
    return _EMB_KERNEL(token_table, input, pe)
